# Initial kernel scaffold; baseline (speedup 1.0000x reference)
import jax
import jax.numpy as jnp
from jax.experimental import pallas as pl


def kernel(features, rois):
    raise NotImplementedError("write your pallas kernel here")


# speedup vs baseline: 1.2094x; 1.2094x over previous
import functools

import jax
import jax.numpy as jnp
from jax import lax
from jax.experimental import pallas as pl
from jax.experimental.pallas import tpu as pltpu
from jax.experimental.pallas import tpu_sc as plsc

C = 192
H = 224
W = 224
OH = 7
OW = 7
N_ROIS = 1000
NSAMP = N_ROIS * OH * OW

NW = 32
S_PER_W = 1536
NPAD = NW * S_PER_W
CHUNK = 32
NCHUNKS = S_PER_W // CHUNK
L = 16
NCH = C // L

_mesh = plsc.VectorSubcoreMesh(core_axis_name="c", subcore_axis_name="s")


@functools.partial(
    pl.kernel,
    out_type=jax.ShapeDtypeStruct((NPAD, C), jnp.float32),
    mesh=_mesh,
    compiler_params=pltpu.CompilerParams(
        needs_layout_passes=False, use_tc_tiling_on_sc=False),
    scratch_types=[
        pltpu.VMEM((4 * N_ROIS,), jnp.float32),
        pltpu.VMEM((S_PER_W,), jnp.int32),
        pltpu.VMEM((S_PER_W,), jnp.float32),
        pltpu.VMEM((S_PER_W,), jnp.float32),
        pltpu.VMEM((4 * CHUNK,), jnp.int32),
        pltpu.VMEM((4 * CHUNK,), jnp.float32),
        pltpu.VMEM((4 * CHUNK, C), jnp.float32),
        pltpu.VMEM((CHUNK, C), jnp.float32),
        pltpu.SemaphoreType.DMA,
    ],
)
def _roi_align_sc(feat_hbm, rois_hbm, nmap_hbm, ohp5_hbm, owp5_hbm, out_hbm,
                  rois_v, nmap_v, ohp5_v, owp5_v, idx_v, w_v, g_v, ob_v, sem):
    wid = lax.axis_index("s") * 2 + lax.axis_index("c")
    base = wid * S_PER_W

    pltpu.sync_copy(rois_hbm, rois_v)
    pltpu.sync_copy(nmap_hbm.at[pl.ds(base, S_PER_W)], nmap_v)
    pltpu.sync_copy(ohp5_hbm.at[pl.ds(base, S_PER_W)], ohp5_v)
    pltpu.sync_copy(owp5_hbm.at[pl.ds(base, S_PER_W)], owp5_v)

    zeros16 = jnp.zeros((L,), jnp.int32)

    def chunk_body(ch, _):
        for g in range(CHUNK // L):
            off = ch * CHUNK + g * L
            n = nmap_v[pl.ds(off, L)]
            ohv = ohp5_v[pl.ds(off, L)]
            owv = owp5_v[pl.ds(off, L)]
            n4 = n * 4
            x1 = plsc.load_gather(rois_v, [n4]) * float(W)
            y1 = plsc.load_gather(rois_v, [n4 + 1]) * float(H)
            x2 = plsc.load_gather(rois_v, [n4 + 2]) * float(W)
            y2 = plsc.load_gather(rois_v, [n4 + 3]) * float(H)
            bh = (y2 - y1) / float(OH)
            bw = (x2 - x1) / float(OW)
            y = jnp.clip(y1 + ohv * bh, 0.0, float(H - 1))
            x = jnp.clip(x1 + owv * bw, 0.0, float(W - 1))
            y0 = y.astype(jnp.int32)
            x0 = x.astype(jnp.int32)
            wy = y - y0.astype(jnp.float32)
            wx = x - x0.astype(jnp.float32)
            y1i = jnp.minimum(y0 + 1, H - 1)
            x1i = jnp.minimum(x0 + 1, W - 1)
            valid = jnp.where((bh > 0.0) & (bw > 0.0), 1.0, 0.0)
            omwx = 1.0 - wx
            omwy = 1.0 - wy
            yb0 = y0 * W
            yb1 = y1i * W
            idx_v[pl.ds(0 * CHUNK + g * L, L)] = yb0 + x0
            idx_v[pl.ds(1 * CHUNK + g * L, L)] = yb0 + x1i
            idx_v[pl.ds(2 * CHUNK + g * L, L)] = yb1 + x0
            idx_v[pl.ds(3 * CHUNK + g * L, L)] = yb1 + x1i
            w_v[pl.ds(0 * CHUNK + g * L, L)] = omwx * omwy * valid
            w_v[pl.ds(1 * CHUNK + g * L, L)] = wx * omwy * valid
            w_v[pl.ds(2 * CHUNK + g * L, L)] = omwx * wy * valid
            w_v[pl.ds(3 * CHUNK + g * L, L)] = wx * wy * valid

        pltpu.async_copy(feat_hbm.at[idx_v], g_v, sem).wait()

        def samp_body(s, _):
            w00 = plsc.load_gather(w_v, [zeros16 + (0 * CHUNK) + s])
            w01 = plsc.load_gather(w_v, [zeros16 + (1 * CHUNK) + s])
            w10 = plsc.load_gather(w_v, [zeros16 + (2 * CHUNK) + s])
            w11 = plsc.load_gather(w_v, [zeros16 + (3 * CHUNK) + s])
            for j in range(NCH):
                cs = pl.ds(j * L, L)
                acc = (g_v[0 * CHUNK + s, cs] * w00
                       + g_v[1 * CHUNK + s, cs] * w01
                       + g_v[2 * CHUNK + s, cs] * w10
                       + g_v[3 * CHUNK + s, cs] * w11)
                ob_v[s, cs] = acc
            return 0

        lax.fori_loop(0, CHUNK, samp_body, 0)
        pltpu.sync_copy(ob_v, out_hbm.at[pl.ds(base + ch * CHUNK, CHUNK)])
        return 0

    lax.fori_loop(0, NCHUNKS, chunk_body, 0)


def kernel(features, rois):
    feat_t = jnp.transpose(features, (1, 2, 0)).reshape(H * W, C)
    s = jnp.arange(NPAD, dtype=jnp.int32)
    n_map = jnp.minimum(s // (OH * OW), N_ROIS - 1)
    b = s % (OH * OW)
    ohp5 = (b // OW).astype(jnp.float32) + 0.5
    owp5 = (b % OW).astype(jnp.float32) + 0.5
    out_flat = _roi_align_sc(feat_t, rois.reshape(-1), n_map, ohp5, owp5)
    out = out_flat[:NSAMP].reshape(N_ROIS, OH * OW, C)
    return jnp.transpose(out, (0, 2, 1)).reshape(N_ROIS, C, OH, OW)
